# initial kernel scaffold (unmeasured)
import jax
import jax.numpy as jnp
from jax import lax
from jax.experimental import pallas as pl
from jax.experimental.pallas import tpu as pltpu

_BITS = (2048, 1024, 512, 256, 128, 64, 32, 16, 8, 4, 2, 1)


def kernel(x, dest):
    n, d = x.shape
    r = lax.axis_index("x")

    order = jnp.argsort(dest, stable=True)
    xs = x.astype(jnp.bfloat16)[order]
    k = jnp.sum((dest != r).astype(jnp.int32)).reshape(1)

    def body(k_ref, xs_ref, out_ref, send_sems, recv_sems, local_sem):
        rr = lax.axis_index("x")
        yy = lax.axis_index("y")
        zz = lax.axis_index("z")
        peer = (1 - rr, yy, zz)
        K = k_ref[0]
        n_own = n - K
        is0 = rr == 0
        off_send = jnp.where(is0, n_own, 0)
        dst_off = jnp.where(is0, 0, n_own)
        own_off = jnp.where(is0, 0, K)
        recv_off = jnp.where(is0, n_own, 0)

        bsem = pltpu.get_barrier_semaphore()
        pl.semaphore_signal(
            bsem, inc=1, device_id=peer, device_id_type=pl.DeviceIdType.MESH
        )
        pl.semaphore_wait(bsem, 1)

        prefix = jnp.int32(0)
        for i, bit in enumerate(_BITS):
            @pl.when((K & bit) != 0)
            def _(i=i, bit=bit, p=prefix):
                pltpu.make_async_remote_copy(
                    src_ref=xs_ref.at[pl.ds(off_send + p, bit)],
                    dst_ref=out_ref.at[pl.ds(dst_off + p, bit)],
                    send_sem=send_sems.at[i],
                    recv_sem=recv_sems.at[i],
                    device_id=peer,
                    device_id_type=pl.DeviceIdType.MESH,
                ).start()
            prefix = prefix + (K & bit)

        prefix = jnp.int32(0)
        for bit in _BITS:
            @pl.when((n_own & bit) != 0)
            def _(bit=bit, p=prefix):
                cp = pltpu.make_async_copy(
                    xs_ref.at[pl.ds(own_off + p, bit)],
                    out_ref.at[pl.ds(own_off + p, bit)],
                    local_sem,
                )
                cp.start()
                cp.wait()
            prefix = prefix + (n_own & bit)

        prefix = jnp.int32(0)
        for i, bit in enumerate(_BITS):
            @pl.when((K & bit) != 0)
            def _(i=i, bit=bit, p=prefix):
                pltpu.make_async_remote_copy(
                    src_ref=xs_ref.at[pl.ds(0, bit)],
                    dst_ref=out_ref.at[pl.ds(recv_off + p, bit)],
                    send_sem=send_sems.at[i],
                    recv_sem=recv_sems.at[i],
                    device_id=peer,
                    device_id_type=pl.DeviceIdType.MESH,
                ).wait_recv()
            prefix = prefix + (K & bit)

        prefix = jnp.int32(0)
        for i, bit in enumerate(_BITS):
            @pl.when((K & bit) != 0)
            def _(i=i, bit=bit, p=prefix):
                pltpu.make_async_remote_copy(
                    src_ref=xs_ref.at[pl.ds(off_send + p, bit)],
                    dst_ref=out_ref.at[pl.ds(dst_off + p, bit)],
                    send_sem=send_sems.at[i],
                    recv_sem=recv_sems.at[i],
                    device_id=peer,
                    device_id_type=pl.DeviceIdType.MESH,
                ).wait_send()
            prefix = prefix + (K & bit)

    return pl.pallas_call(
        body,
        out_shape=jax.ShapeDtypeStruct((n, d), jnp.bfloat16),
        in_specs=[
            pl.BlockSpec(memory_space=pltpu.SMEM),
            pl.BlockSpec(memory_space=pltpu.VMEM),
        ],
        out_specs=pl.BlockSpec(memory_space=pltpu.VMEM),
        scratch_shapes=[
            pltpu.SemaphoreType.DMA((len(_BITS),)),
            pltpu.SemaphoreType.DMA((len(_BITS),)),
            pltpu.SemaphoreType.DMA,
        ],
        compiler_params=pltpu.CompilerParams(collective_id=0),
    )(k, xs)


# baseline (device time: 50587 ns/iter reference)
import jax
import jax.numpy as jnp
from jax import lax
from jax.experimental import pallas as pl
from jax.experimental.pallas import tpu as pltpu

_BITS = (2048, 1024, 512, 256, 128, 64, 32, 16, 8, 4, 2, 1)


def kernel(x, dest):
    n, d = x.shape
    r = lax.axis_index("x")

    order = jnp.argsort(dest, stable=True)
    xs = x.astype(jnp.bfloat16)[order].reshape(n, d // 128, 128)
    k = jnp.sum((dest != r).astype(jnp.int32)).reshape(1)

    def body(k_ref, xs_ref, out_ref, send_sems, recv_sems, local_sem):
        rr = lax.axis_index("x")
        yy = lax.axis_index("y")
        zz = lax.axis_index("z")
        peer = (1 - rr, yy, zz)
        K = k_ref[0]
        n_own = n - K
        is0 = rr == 0
        off_send = jnp.where(is0, n_own, 0)
        dst_off = jnp.where(is0, 0, n_own)
        own_off = jnp.where(is0, 0, K)
        recv_off = jnp.where(is0, n_own, 0)

        bsem = pltpu.get_barrier_semaphore()
        pl.semaphore_signal(
            bsem, inc=1, device_id=peer, device_id_type=pl.DeviceIdType.MESH
        )
        pl.semaphore_wait(bsem, 1)

        prefix = jnp.int32(0)
        for i, bit in enumerate(_BITS):
            @pl.when((K & bit) != 0)
            def _(i=i, bit=bit, p=prefix):
                pltpu.make_async_remote_copy(
                    src_ref=xs_ref.at[pl.ds(off_send + p, bit)],
                    dst_ref=out_ref.at[pl.ds(dst_off + p, bit)],
                    send_sem=send_sems.at[i],
                    recv_sem=recv_sems.at[i],
                    device_id=peer,
                    device_id_type=pl.DeviceIdType.MESH,
                ).start()
            prefix = prefix + (K & bit)

        prefix = jnp.int32(0)
        for bit in _BITS:
            @pl.when((n_own & bit) != 0)
            def _(bit=bit, p=prefix):
                cp = pltpu.make_async_copy(
                    xs_ref.at[pl.ds(own_off + p, bit)],
                    out_ref.at[pl.ds(own_off + p, bit)],
                    local_sem,
                )
                cp.start()
                cp.wait()
            prefix = prefix + (n_own & bit)

        prefix = jnp.int32(0)
        for i, bit in enumerate(_BITS):
            @pl.when((K & bit) != 0)
            def _(i=i, bit=bit, p=prefix):
                pltpu.make_async_remote_copy(
                    src_ref=xs_ref.at[pl.ds(0, bit)],
                    dst_ref=out_ref.at[pl.ds(recv_off + p, bit)],
                    send_sem=send_sems.at[i],
                    recv_sem=recv_sems.at[i],
                    device_id=peer,
                    device_id_type=pl.DeviceIdType.MESH,
                ).wait_recv()
            prefix = prefix + (K & bit)

        prefix = jnp.int32(0)
        for i, bit in enumerate(_BITS):
            @pl.when((K & bit) != 0)
            def _(i=i, bit=bit, p=prefix):
                pltpu.make_async_remote_copy(
                    src_ref=xs_ref.at[pl.ds(off_send + p, bit)],
                    dst_ref=out_ref.at[pl.ds(dst_off + p, bit)],
                    send_sem=send_sems.at[i],
                    recv_sem=recv_sems.at[i],
                    device_id=peer,
                    device_id_type=pl.DeviceIdType.MESH,
                ).wait_send()
            prefix = prefix + (K & bit)

    out = pl.pallas_call(
        body,
        out_shape=jax.ShapeDtypeStruct((n, d // 128, 128), jnp.bfloat16),
        in_specs=[
            pl.BlockSpec(memory_space=pltpu.SMEM),
            pl.BlockSpec(memory_space=pltpu.VMEM),
        ],
        out_specs=pl.BlockSpec(memory_space=pltpu.VMEM),
        scratch_shapes=[
            pltpu.SemaphoreType.DMA((len(_BITS),)),
            pltpu.SemaphoreType.DMA((len(_BITS),)),
            pltpu.SemaphoreType.DMA,
        ],
        compiler_params=pltpu.CompilerParams(collective_id=0),
    )(k, xs)
    return out.reshape(n, d)


# device time: 48707 ns/iter; 1.0386x vs baseline; 1.0386x over previous
import jax
import jax.numpy as jnp
from jax import lax
from jax.experimental import pallas as pl
from jax.experimental.pallas import tpu as pltpu

_BITS = (2048, 1024, 512, 256, 128, 64, 32, 16, 8, 4, 2, 1)


def kernel(x, dest):
    n, d = x.shape
    r = lax.axis_index("x")

    order = jnp.argsort(dest, stable=True)
    xs = x.reshape(n, d // 128, 128).astype(jnp.bfloat16)[order]
    k = jnp.sum((dest != r).astype(jnp.int32)).reshape(1)

    def body(k_ref, xs_ref, out_ref, send_sems, recv_sems, local_sem):
        rr = lax.axis_index("x")
        yy = lax.axis_index("y")
        zz = lax.axis_index("z")
        peer = (1 - rr, yy, zz)
        K = k_ref[0]
        n_own = n - K
        is0 = rr == 0
        off_send = jnp.where(is0, n_own, 0)
        dst_off = jnp.where(is0, 0, n_own)
        own_off = jnp.where(is0, 0, K)
        recv_off = jnp.where(is0, n_own, 0)

        bsem = pltpu.get_barrier_semaphore()
        pl.semaphore_signal(
            bsem, inc=1, device_id=peer, device_id_type=pl.DeviceIdType.MESH
        )
        pl.semaphore_wait(bsem, 1)

        prefix = jnp.int32(0)
        for i, bit in enumerate(_BITS):
            @pl.when((K & bit) != 0)
            def _(i=i, bit=bit, p=prefix):
                pltpu.make_async_remote_copy(
                    src_ref=xs_ref.at[pl.ds(off_send + p, bit)],
                    dst_ref=out_ref.at[pl.ds(dst_off + p, bit)],
                    send_sem=send_sems.at[i],
                    recv_sem=recv_sems.at[i],
                    device_id=peer,
                    device_id_type=pl.DeviceIdType.MESH,
                ).start()
            prefix = prefix + (K & bit)

        prefix = jnp.int32(0)
        for bit in _BITS:
            @pl.when((n_own & bit) != 0)
            def _(bit=bit, p=prefix):
                pltpu.make_async_copy(
                    xs_ref.at[pl.ds(own_off + p, bit)],
                    out_ref.at[pl.ds(own_off + p, bit)],
                    local_sem,
                ).start()
            prefix = prefix + (n_own & bit)
        prefix = jnp.int32(0)
        for bit in _BITS:
            @pl.when((n_own & bit) != 0)
            def _(bit=bit, p=prefix):
                pltpu.make_async_copy(
                    xs_ref.at[pl.ds(own_off + p, bit)],
                    out_ref.at[pl.ds(own_off + p, bit)],
                    local_sem,
                ).wait()
            prefix = prefix + (n_own & bit)

        prefix = jnp.int32(0)
        for i, bit in enumerate(_BITS):
            @pl.when((K & bit) != 0)
            def _(i=i, bit=bit, p=prefix):
                pltpu.make_async_remote_copy(
                    src_ref=xs_ref.at[pl.ds(0, bit)],
                    dst_ref=out_ref.at[pl.ds(recv_off + p, bit)],
                    send_sem=send_sems.at[i],
                    recv_sem=recv_sems.at[i],
                    device_id=peer,
                    device_id_type=pl.DeviceIdType.MESH,
                ).wait_recv()
            prefix = prefix + (K & bit)

        prefix = jnp.int32(0)
        for i, bit in enumerate(_BITS):
            @pl.when((K & bit) != 0)
            def _(i=i, bit=bit, p=prefix):
                pltpu.make_async_remote_copy(
                    src_ref=xs_ref.at[pl.ds(off_send + p, bit)],
                    dst_ref=out_ref.at[pl.ds(dst_off + p, bit)],
                    send_sem=send_sems.at[i],
                    recv_sem=recv_sems.at[i],
                    device_id=peer,
                    device_id_type=pl.DeviceIdType.MESH,
                ).wait_send()
            prefix = prefix + (K & bit)

    out = pl.pallas_call(
        body,
        out_shape=jax.ShapeDtypeStruct((n, d // 128, 128), jnp.bfloat16),
        in_specs=[
            pl.BlockSpec(memory_space=pltpu.SMEM),
            pl.BlockSpec(memory_space=pltpu.VMEM),
        ],
        out_specs=pl.BlockSpec(memory_space=pltpu.VMEM),
        scratch_shapes=[
            pltpu.SemaphoreType.DMA((len(_BITS),)),
            pltpu.SemaphoreType.DMA((len(_BITS),)),
            pltpu.SemaphoreType.DMA,
        ],
        compiler_params=pltpu.CompilerParams(collective_id=0),
    )(k, xs)
    return out.reshape(n, d)
